# baseline (speedup 1.0000x reference)
import functools

import jax
import jax.numpy as jnp
from jax import lax
from jax.experimental import pallas as pl
from jax.experimental.pallas import tpu as pltpu
from jax.experimental.pallas import tpu_sc as plsc

_NC = 2
_NS = 16
_NW = _NC * _NS
_CH = 128


def _make_agg(N, E, D):
    n_chunks = E // _CH
    cpw = n_chunks // _NW
    extra = n_chunks - cpw * _NW
    CR = 80
    n_row_chunks = N // CR
    passes = (n_row_chunks + _NS - 1) // _NS
    mesh = plsc.VectorSubcoreMesh(core_axis_name="c", subcore_axis_name="s")

    @functools.partial(
        pl.kernel,
        out_type=jax.ShapeDtypeStruct((_NC, N, D), jnp.float32),
        mesh=mesh,
        scratch_types=[
            pltpu.VMEM((3, _CH), jnp.int32),
            pltpu.VMEM((3, _CH), jnp.int32),
            pltpu.VMEM((_CH, D), jnp.float32),
            pltpu.VMEM((_CH, D), jnp.float32),
            pltpu.VMEM((CR, D), jnp.float32),
            pltpu.VMEM_SHARED((N, D), jnp.float32),
            pltpu.SemaphoreType.DMA,
            pltpu.SemaphoreType.DMA,
            pltpu.SemaphoreType.DMA,
            pltpu.SemaphoreType.DMA,
            pltpu.SemaphoreType.DMA,
            pltpu.SemaphoreType.DMA,
            pltpu.SemaphoreType.DMA,
        ],
    )
    def agg(x_hbm, src_hbm, dst_hbm, zeros_hbm, out_hbm,
            srci, dsti, rows0, rows1, stage_v, acc_sh,
            gsem0, gsem1, ssem0, ssem1, isem0, isem1, isem2):
        cid = lax.axis_index("c")
        sid = lax.axis_index("s")
        wid = sid * _NC + cid

        pltpu.sync_copy(zeros_hbm, stage_v)
        for t in range(passes):
            ch = t * _NS + sid
            @pl.when(ch < n_row_chunks)
            def _():
                pltpu.sync_copy(stage_v, acc_sh.at[pl.ds(ch * CR, CR)])

        plsc.subcore_barrier()

        rows = (rows0, rows1)
        gsem = (gsem0, gsem1)
        ssem = (ssem0, ssem1)
        isem = (isem0, isem1, isem2)
        e0 = wid * cpw

        def idx_copies(chunk, slot):
            off = (e0 + chunk) * _CH
            return (pltpu.make_async_copy(src_hbm.at[pl.ds(off, _CH)],
                                          srci.at[slot], isem[slot]),
                    pltpu.make_async_copy(dst_hbm.at[pl.ds(off, _CH)],
                                          dsti.at[slot], isem[slot]))

        def gather(chunk, slot, b):
            return pltpu.make_async_copy(x_hbm.at[srci.at[slot]], rows[b], gsem[b])

        def scatter(chunk, slot, b):
            return pltpu.make_async_copy(rows[b], acc_sh.at[dsti.at[slot]], ssem[b])

        for c in idx_copies(0, 0):
            c.start()
            c.wait()
        gather(0, 0, 0).start()
        for c in idx_copies(1, 1):
            c.start()
        for c in idx_copies(2, 2):
            c.start()

        @pl.loop(0, cpw // 6)
        def _edges(go):
            for u in range(6):
                g = 6 * go + u
                b = u % 2
                i3 = u % 3

                gather(g, i3, b).wait()

                @pl.when(g + 1 < cpw)
                def _():
                    for c in idx_copies(g + 1, (u + 1) % 3):
                        c.wait()

                    @pl.when(g >= 1)
                    def _():
                        scatter(g - 1, (u + 2) % 3, 1 - b).wait()

                    gather(g + 1, (u + 1) % 3, 1 - b).start()

                scatter(g, i3, b).start(add=True)

                @pl.when(g + 2 < cpw)
                def _():
                    for c in idx_copies(g + 2, (u + 2) % 3):
                        c.start()

        scatter(cpw - 2, (cpw - 2) % 3, 0).wait()
        scatter(cpw - 1, (cpw - 1) % 3, 1).wait()

        @pl.when(wid < extra)
        def _tail():
            off = (_NW * cpw + wid) * _CH
            pltpu.sync_copy(src_hbm.at[pl.ds(off, _CH)], srci.at[0])
            pltpu.sync_copy(dst_hbm.at[pl.ds(off, _CH)], dsti.at[0])
            pltpu.async_copy(x_hbm.at[srci.at[0]], rows0, gsem0).wait()
            pltpu.sync_copy(rows0, acc_sh.at[dsti.at[0]], add=True)

        plsc.subcore_barrier()

        for t in range(passes):
            ch = t * _NS + sid
            @pl.when(ch < n_row_chunks)
            def _():
                pltpu.sync_copy(acc_sh.at[pl.ds(ch * CR, CR)], stage_v)
                pltpu.sync_copy(stage_v, out_hbm.at[cid, pl.ds(ch * CR, CR)])

    return agg


def _mlp_body(x_ref, p0_ref, p1_ref, w1_ref, b1_ref, w2_ref, b2_ref,
              h_ref, sum_ref, sq_ref):
    i = pl.program_id(0)
    s = x_ref[...] + p0_ref[...] + p1_ref[...]
    h = jnp.maximum(
        jnp.dot(s, w1_ref[...], preferred_element_type=jnp.float32) + b1_ref[...], 0.0)
    h = jnp.maximum(
        jnp.dot(h, w2_ref[...], preferred_element_type=jnp.float32) + b2_ref[...], 0.0)
    h_ref[...] = h

    @pl.when(i == 0)
    def _():
        sum_ref[...] = jnp.zeros_like(sum_ref)
        sq_ref[...] = jnp.zeros_like(sq_ref)

    sum_ref[0:1, :] = sum_ref[0:1, :] + jnp.sum(h, axis=0, keepdims=True)
    sq_ref[0:1, :] = sq_ref[0:1, :] + jnp.sum(h * h, axis=0, keepdims=True)


def _bn_body(n_rows, h_ref, sum_ref, sq_ref, g_ref, b_ref, o_ref):
    mean = sum_ref[0:1, :] / n_rows
    var = sq_ref[0:1, :] / n_rows - mean * mean
    rstd = lax.rsqrt(var + 1e-5)
    o_ref[...] = (h_ref[...] - mean) * rstd * g_ref[...] + b_ref[...]


def kernel(x, edge_index, gamma, beta,
           W1_0, b1_0, W2_0, b2_0,
           W1_1, b1_1, W2_1, b2_1,
           W1_2, b1_2, W2_2, b2_2):
    N, D = x.shape
    E = edge_index.shape[1]
    H = W1_0.shape[1]
    n_chunks = E // _CH
    cpw = n_chunks // _NW
    n_main = _NW * cpw

    src_1d = edge_index[0]
    dst_1d = edge_index[1]
    zeros_h = jnp.zeros((80, D), jnp.float32)

    agg = _make_agg(N, E, D)

    NB = 10
    B = N // NB
    row_spec = pl.BlockSpec((B, D), lambda i: (i, 0))
    full = lambda shape: pl.BlockSpec(shape, lambda i: (0, 0))

    mlp_call = pl.pallas_call(
        _mlp_body,
        grid=(NB,),
        in_specs=[row_spec, row_spec, row_spec,
                  full((D, H)), full((1, H)), full((H, H)), full((1, H))],
        out_specs=[pl.BlockSpec((B, H), lambda i: (i, 0)),
                   full((8, H)), full((8, H))],
        out_shape=[jax.ShapeDtypeStruct((N, H), jnp.float32),
                   jax.ShapeDtypeStruct((8, H), jnp.float32),
                   jax.ShapeDtypeStruct((8, H), jnp.float32)],
    )
    bn_call = pl.pallas_call(
        functools.partial(_bn_body, float(N)),
        grid=(NB,),
        in_specs=[pl.BlockSpec((B, H), lambda i: (i, 0)),
                  full((8, H)), full((8, H)), full((1, H)), full((1, H))],
        out_specs=pl.BlockSpec((B, H), lambda i: (i, 0)),
        out_shape=jax.ShapeDtypeStruct((N, H), jnp.float32),
    )

    g_r = gamma.reshape(1, H)
    be_r = beta.reshape(1, H)
    params = [(W1_0, b1_0, W2_0, b2_0),
              (W1_1, b1_1, W2_1, b2_1),
              (W1_2, b1_2, W2_2, b2_2)]

    cur = x
    for (W1, b1, W2, b2) in params:
        parts = agg(cur, src_1d, dst_1d, zeros_h)
        h, ssum, ssq = mlp_call(cur, parts[0], parts[1],
                                W1, b1.reshape(1, H), W2, b2.reshape(1, H))
        cur = bn_call(h, ssum, ssq, g_r, be_r)
    return cur


# speedup vs baseline: 1.1471x; 1.1471x over previous
import functools

import jax
import jax.numpy as jnp
from jax import lax
from jax.experimental import pallas as pl
from jax.experimental.pallas import tpu as pltpu
from jax.experimental.pallas import tpu_sc as plsc

_NC = 2
_NS = 16
_NW = _NC * _NS
_CH = 64
_CR = 40


def _make_agg(N, E, D):
    n_chunks = E // _CH
    cpw = n_chunks // _NW
    extra = n_chunks - cpw * _NW
    CR = _CR
    n_row_chunks = N // CR
    passes = (n_row_chunks + _NS - 1) // _NS
    mesh = plsc.VectorSubcoreMesh(core_axis_name="c", subcore_axis_name="s")

    @functools.partial(
        pl.kernel,
        out_type=jax.ShapeDtypeStruct((_NC, N, D), jnp.float32),
        mesh=mesh,
        scratch_types=[
            pltpu.VMEM((12, _CH), jnp.int32),
            pltpu.VMEM((_CH, D), jnp.float32),
            pltpu.VMEM((_CH, D), jnp.float32),
            pltpu.VMEM((_CH, D), jnp.float32),
            pltpu.VMEM_SHARED((N, D), jnp.float32),
            [pltpu.SemaphoreType.DMA] * 3,
            [pltpu.SemaphoreType.DMA] * 3,
            [pltpu.SemaphoreType.DMA] * 6,
        ],
    )
    def agg(x_hbm, src_hbm, dst_hbm, zeros_hbm, out_hbm,
            idxr, rows0, rows1, rows2, acc_sh, gsem, ssem, isem):
        cid = lax.axis_index("c")
        sid = lax.axis_index("s")
        wid = sid * _NC + cid
        rows = (rows0, rows1, rows2)
        stage = rows0.at[pl.ds(0, CR)]

        pltpu.sync_copy(zeros_hbm, stage)
        for t in range(passes):
            ch = t * _NS + sid
            @pl.when(ch < n_row_chunks)
            def _():
                pltpu.sync_copy(stage, acc_sh.at[pl.ds(ch * CR, CR)])

        plsc.subcore_barrier()

        e0 = wid * cpw

        def idx_copies(chunk, slot):
            off = (e0 + chunk) * _CH
            return (pltpu.make_async_copy(src_hbm.at[pl.ds(off, _CH)],
                                          idxr.at[slot], isem[slot]),
                    pltpu.make_async_copy(dst_hbm.at[pl.ds(off, _CH)],
                                          idxr.at[6 + slot], isem[slot]))

        def gather(slot6, slot3):
            return pltpu.make_async_copy(x_hbm.at[idxr.at[slot6]],
                                         rows[slot3], gsem[slot3])

        def scatter(slot6, slot3):
            return pltpu.make_async_copy(rows[slot3],
                                         acc_sh.at[idxr.at[6 + slot6]], ssem[slot3])

        for s in (0, 1):
            for c in idx_copies(s, s):
                c.start()
        for s in (0, 1):
            for c in idx_copies(s, s):
                c.wait()
            gather(s, s).start()
        for s in (2, 3, 4):
            for c in idx_copies(s, s):
                c.start()

        @pl.loop(0, cpw // 6)
        def _edges(go):
            for u in range(6):
                g = 6 * go + u
                s3 = u % 3

                gather(u, s3).wait()
                scatter(u, s3).start(add=True)

                @pl.when(g >= 1)
                def _():
                    scatter((u + 5) % 6, (u + 2) % 3).wait()

                @pl.when(g + 2 < cpw)
                def _():
                    for c in idx_copies(g + 2, (u + 2) % 6):
                        c.wait()
                    gather((u + 2) % 6, (u + 2) % 3).start()

                @pl.when(g + 5 < cpw)
                def _():
                    for c in idx_copies(g + 5, (u + 5) % 6):
                        c.start()

        scatter((cpw - 1) % 6, (cpw - 1) % 3).wait()

        @pl.when(wid < extra)
        def _tail():
            off = (_NW * cpw + wid) * _CH
            pltpu.sync_copy(src_hbm.at[pl.ds(off, _CH)], idxr.at[0])
            pltpu.sync_copy(dst_hbm.at[pl.ds(off, _CH)], idxr.at[6])
            pltpu.async_copy(x_hbm.at[idxr.at[0]], rows0, gsem[0]).wait()
            pltpu.sync_copy(rows0, acc_sh.at[idxr.at[6]], add=True)

        plsc.subcore_barrier()

        for t in range(passes):
            ch = t * _NS + sid
            @pl.when(ch < n_row_chunks)
            def _():
                pltpu.sync_copy(acc_sh.at[pl.ds(ch * CR, CR)], stage)
                pltpu.sync_copy(stage, out_hbm.at[cid, pl.ds(ch * CR, CR)])

    return agg


def _mlp_body(x_ref, p0_ref, p1_ref, w1_ref, b1_ref, w2_ref, b2_ref,
              h_ref, sum_ref, sq_ref):
    i = pl.program_id(0)
    s = x_ref[...] + p0_ref[...] + p1_ref[...]
    h = jnp.maximum(
        jnp.dot(s, w1_ref[...], preferred_element_type=jnp.float32) + b1_ref[...], 0.0)
    h = jnp.maximum(
        jnp.dot(h, w2_ref[...], preferred_element_type=jnp.float32) + b2_ref[...], 0.0)
    h_ref[...] = h

    @pl.when(i == 0)
    def _():
        sum_ref[...] = jnp.zeros_like(sum_ref)
        sq_ref[...] = jnp.zeros_like(sq_ref)

    sum_ref[0:1, :] = sum_ref[0:1, :] + jnp.sum(h, axis=0, keepdims=True)
    sq_ref[0:1, :] = sq_ref[0:1, :] + jnp.sum(h * h, axis=0, keepdims=True)


def _bn_body(n_rows, h_ref, sum_ref, sq_ref, g_ref, b_ref, o_ref):
    mean = sum_ref[0:1, :] / n_rows
    var = sq_ref[0:1, :] / n_rows - mean * mean
    rstd = lax.rsqrt(var + 1e-5)
    o_ref[...] = (h_ref[...] - mean) * rstd * g_ref[...] + b_ref[...]


def kernel(x, edge_index, gamma, beta,
           W1_0, b1_0, W2_0, b2_0,
           W1_1, b1_1, W2_1, b2_1,
           W1_2, b1_2, W2_2, b2_2):
    N, D = x.shape
    E = edge_index.shape[1]
    H = W1_0.shape[1]
    n_chunks = E // _CH
    cpw = n_chunks // _NW
    n_main = _NW * cpw

    src_1d = edge_index[0]
    dst_1d = edge_index[1]
    zeros_h = jnp.zeros((_CR, D), jnp.float32)

    agg = _make_agg(N, E, D)

    NB = 10
    B = N // NB
    row_spec = pl.BlockSpec((B, D), lambda i: (i, 0))
    full = lambda shape: pl.BlockSpec(shape, lambda i: (0, 0))

    mlp_call = pl.pallas_call(
        _mlp_body,
        grid=(NB,),
        in_specs=[row_spec, row_spec, row_spec,
                  full((D, H)), full((1, H)), full((H, H)), full((1, H))],
        out_specs=[pl.BlockSpec((B, H), lambda i: (i, 0)),
                   full((8, H)), full((8, H))],
        out_shape=[jax.ShapeDtypeStruct((N, H), jnp.float32),
                   jax.ShapeDtypeStruct((8, H), jnp.float32),
                   jax.ShapeDtypeStruct((8, H), jnp.float32)],
    )
    bn_call = pl.pallas_call(
        functools.partial(_bn_body, float(N)),
        grid=(NB,),
        in_specs=[pl.BlockSpec((B, H), lambda i: (i, 0)),
                  full((8, H)), full((8, H)), full((1, H)), full((1, H))],
        out_specs=pl.BlockSpec((B, H), lambda i: (i, 0)),
        out_shape=jax.ShapeDtypeStruct((N, H), jnp.float32),
    )

    g_r = gamma.reshape(1, H)
    be_r = beta.reshape(1, H)
    params = [(W1_0, b1_0, W2_0, b2_0),
              (W1_1, b1_1, W2_1, b2_1),
              (W1_2, b1_2, W2_2, b2_2)]

    cur = x
    for (W1, b1, W2, b2) in params:
        parts = agg(cur, src_1d, dst_1d, zeros_h)
        h, ssum, ssq = mlp_call(cur, parts[0], parts[1],
                                W1, b1.reshape(1, H), W2, b2.reshape(1, H))
        cur = bn_call(h, ssum, ssq, g_r, be_r)
    return cur


# speedup vs baseline: 1.1905x; 1.0378x over previous
import functools

import jax
import jax.numpy as jnp
from jax import lax
from jax.experimental import pallas as pl
from jax.experimental.pallas import tpu as pltpu
from jax.experimental.pallas import tpu_sc as plsc

_NC = 2
_NS = 16
_NW = _NC * _NS
_CH = 32
_CR = 16
_NR = 6
_NI = 12


def _make_agg(N, E, D):
    n_chunks = E // _CH
    cpw = n_chunks // _NW
    extra = n_chunks - cpw * _NW
    CR = _CR
    n_row_chunks = N // CR
    passes = (n_row_chunks + _NS - 1) // _NS
    mesh = plsc.VectorSubcoreMesh(core_axis_name="c", subcore_axis_name="s")

    @functools.partial(
        pl.kernel,
        out_type=jax.ShapeDtypeStruct((_NC, N, D), jnp.float32),
        mesh=mesh,
        scratch_types=[
            pltpu.VMEM((2 * _NI, _CH), jnp.int32),
            [pltpu.VMEM((_CH, D), jnp.float32)] * _NR,
            pltpu.VMEM_SHARED((N, D), jnp.float32),
            [pltpu.SemaphoreType.DMA] * _NR,
            [pltpu.SemaphoreType.DMA] * _NR,
            [pltpu.SemaphoreType.DMA] * _NI,
        ],
    )
    def agg(x_hbm, src_hbm, dst_hbm, zeros_hbm, out_hbm,
            idxr, rows, acc_sh, gsem, ssem, isem):
        cid = lax.axis_index("c")
        sid = lax.axis_index("s")
        wid = sid * _NC + cid
        stage = rows[0].at[pl.ds(0, CR)]

        pltpu.sync_copy(zeros_hbm, stage)
        for t in range(passes):
            ch = t * _NS + sid
            @pl.when(ch < n_row_chunks)
            def _():
                pltpu.sync_copy(stage, acc_sh.at[pl.ds(ch * CR, CR)])

        plsc.subcore_barrier()

        e0 = wid * cpw
        GL = _NR - 1
        IL = _NI - 1

        def idx_copies(chunk, si):
            off = (e0 + chunk) * _CH
            return (pltpu.make_async_copy(src_hbm.at[pl.ds(off, _CH)],
                                          idxr.at[si], isem[si]),
                    pltpu.make_async_copy(dst_hbm.at[pl.ds(off, _CH)],
                                          idxr.at[_NI + si], isem[si]))

        def gather(si, sr):
            return pltpu.make_async_copy(x_hbm.at[idxr.at[si]],
                                         rows[sr], gsem[sr])

        def scatter(si, sr):
            return pltpu.make_async_copy(rows[sr],
                                         acc_sh.at[idxr.at[_NI + si]], ssem[sr])

        for s in range(GL):
            for c in idx_copies(s, s):
                c.start()
        for s in range(GL):
            for c in idx_copies(s, s):
                c.wait()
            gather(s, s % _NR).start()
        for s in range(GL, IL):
            for c in idx_copies(s, s):
                c.start()

        @pl.loop(0, cpw // _NI)
        def _edges(go):
            for u in range(_NI):
                g = _NI * go + u
                sr = u % _NR

                gather(u, sr).wait()
                scatter(u, sr).start(add=True)

                @pl.when(g >= 1)
                def _():
                    scatter((u - 1) % _NI, (u - 1) % _NR).wait()

                @pl.when(g + GL < cpw)
                def _():
                    for c in idx_copies(g + GL, (u + GL) % _NI):
                        c.wait()
                    gather((u + GL) % _NI, (u + GL) % _NR).start()

                @pl.when(g + IL < cpw)
                def _():
                    for c in idx_copies(g + IL, (u + IL) % _NI):
                        c.start()

        scatter((cpw - 1) % _NI, (cpw - 1) % _NR).wait()

        @pl.when(wid < extra)
        def _tail():
            off = (_NW * cpw + wid) * _CH
            pltpu.sync_copy(src_hbm.at[pl.ds(off, _CH)], idxr.at[0])
            pltpu.sync_copy(dst_hbm.at[pl.ds(off, _CH)], idxr.at[_NI])
            pltpu.async_copy(x_hbm.at[idxr.at[0]], rows[0], gsem[0]).wait()
            pltpu.sync_copy(rows[0], acc_sh.at[idxr.at[_NI]], add=True)

        plsc.subcore_barrier()

        for t in range(passes):
            ch = t * _NS + sid
            @pl.when(ch < n_row_chunks)
            def _():
                pltpu.sync_copy(acc_sh.at[pl.ds(ch * CR, CR)], stage)
                pltpu.sync_copy(stage, out_hbm.at[cid, pl.ds(ch * CR, CR)])

    return agg


def _mlp_body(x_ref, p0_ref, p1_ref, w1_ref, b1_ref, w2_ref, b2_ref,
              h_ref, sum_ref, sq_ref):
    i = pl.program_id(0)
    s = x_ref[...] + p0_ref[...] + p1_ref[...]
    h = jnp.maximum(
        jnp.dot(s, w1_ref[...], preferred_element_type=jnp.float32) + b1_ref[...], 0.0)
    h = jnp.maximum(
        jnp.dot(h, w2_ref[...], preferred_element_type=jnp.float32) + b2_ref[...], 0.0)
    h_ref[...] = h

    @pl.when(i == 0)
    def _():
        sum_ref[...] = jnp.zeros_like(sum_ref)
        sq_ref[...] = jnp.zeros_like(sq_ref)

    sum_ref[0:1, :] = sum_ref[0:1, :] + jnp.sum(h, axis=0, keepdims=True)
    sq_ref[0:1, :] = sq_ref[0:1, :] + jnp.sum(h * h, axis=0, keepdims=True)


def _bn_body(n_rows, h_ref, sum_ref, sq_ref, g_ref, b_ref, o_ref):
    mean = sum_ref[0:1, :] / n_rows
    var = sq_ref[0:1, :] / n_rows - mean * mean
    rstd = lax.rsqrt(var + 1e-5)
    o_ref[...] = (h_ref[...] - mean) * rstd * g_ref[...] + b_ref[...]


def kernel(x, edge_index, gamma, beta,
           W1_0, b1_0, W2_0, b2_0,
           W1_1, b1_1, W2_1, b2_1,
           W1_2, b1_2, W2_2, b2_2):
    N, D = x.shape
    E = edge_index.shape[1]
    H = W1_0.shape[1]
    n_chunks = E // _CH
    cpw = n_chunks // _NW
    n_main = _NW * cpw

    src_1d = edge_index[0]
    dst_1d = edge_index[1]
    zeros_h = jnp.zeros((_CR, D), jnp.float32)

    agg = _make_agg(N, E, D)

    NB = 10
    B = N // NB
    row_spec = pl.BlockSpec((B, D), lambda i: (i, 0))
    full = lambda shape: pl.BlockSpec(shape, lambda i: (0, 0))

    mlp_call = pl.pallas_call(
        _mlp_body,
        grid=(NB,),
        in_specs=[row_spec, row_spec, row_spec,
                  full((D, H)), full((1, H)), full((H, H)), full((1, H))],
        out_specs=[pl.BlockSpec((B, H), lambda i: (i, 0)),
                   full((8, H)), full((8, H))],
        out_shape=[jax.ShapeDtypeStruct((N, H), jnp.float32),
                   jax.ShapeDtypeStruct((8, H), jnp.float32),
                   jax.ShapeDtypeStruct((8, H), jnp.float32)],
    )
    bn_call = pl.pallas_call(
        functools.partial(_bn_body, float(N)),
        grid=(NB,),
        in_specs=[pl.BlockSpec((B, H), lambda i: (i, 0)),
                  full((8, H)), full((8, H)), full((1, H)), full((1, H))],
        out_specs=pl.BlockSpec((B, H), lambda i: (i, 0)),
        out_shape=jax.ShapeDtypeStruct((N, H), jnp.float32),
    )

    g_r = gamma.reshape(1, H)
    be_r = beta.reshape(1, H)
    params = [(W1_0, b1_0, W2_0, b2_0),
              (W1_1, b1_1, W2_1, b2_1),
              (W1_2, b1_2, W2_2, b2_2)]

    cur = x
    for (W1, b1, W2, b2) in params:
        parts = agg(cur, src_1d, dst_1d, zeros_h)
        h, ssum, ssq = mlp_call(cur, parts[0], parts[1],
                                W1, b1.reshape(1, H), W2, b2.reshape(1, H))
        cur = bn_call(h, ssum, ssq, g_r, be_r)
    return cur


# speedup vs baseline: 1.2250x; 1.0290x over previous
import functools

import jax
import jax.numpy as jnp
from jax import lax
from jax.experimental import pallas as pl
from jax.experimental.pallas import tpu as pltpu
from jax.experimental.pallas import tpu_sc as plsc

_NC = 2
_NS = 16
_NW = _NC * _NS
_CH = 32
_CR = 400
_NR = 6
_NI = 12


def _make_agg(N, E, D):
    n_chunks = E // _CH
    cpw = n_chunks // _NW
    extra = n_chunks - cpw * _NW
    CR = _CR
    n_row_chunks = N // CR
    passes = (n_row_chunks + _NS - 1) // _NS
    mesh = plsc.VectorSubcoreMesh(core_axis_name="c", subcore_axis_name="s")

    @functools.partial(
        pl.kernel,
        out_type=jax.ShapeDtypeStruct((_NC, N, D), jnp.float32),
        mesh=mesh,
        scratch_types=[
            pltpu.VMEM((2 * _NI, _CH), jnp.int32),
            [pltpu.VMEM((_CH, D), jnp.float32)] * _NR,
            pltpu.VMEM_SHARED((N, D), jnp.float32),
            [pltpu.SemaphoreType.DMA] * _NR,
            [pltpu.SemaphoreType.DMA] * _NR,
            [pltpu.SemaphoreType.DMA] * _NI,
        ],
    )
    def agg(x_hbm, src_hbm, dst_hbm, zeros_hbm, out_hbm,
            idxr, rows, acc_sh, gsem, ssem, isem):
        cid = lax.axis_index("c")
        sid = lax.axis_index("s")
        wid = sid * _NC + cid

        for t in range(passes):
            ch = t * _NS + sid
            @pl.when(ch < n_row_chunks)
            def _():
                pltpu.sync_copy(zeros_hbm, acc_sh.at[pl.ds(ch * CR, CR)])

        plsc.subcore_barrier()

        e0 = wid * cpw
        GL = _NR - 1
        IL = _NI - 1

        def idx_copies(chunk, si):
            off = (e0 + chunk) * _CH
            return (pltpu.make_async_copy(src_hbm.at[pl.ds(off, _CH)],
                                          idxr.at[si], isem[si]),
                    pltpu.make_async_copy(dst_hbm.at[pl.ds(off, _CH)],
                                          idxr.at[_NI + si], isem[si]))

        def gather(si, sr):
            return pltpu.make_async_copy(x_hbm.at[idxr.at[si]],
                                         rows[sr], gsem[sr])

        def scatter(si, sr):
            return pltpu.make_async_copy(rows[sr],
                                         acc_sh.at[idxr.at[_NI + si]], ssem[sr])

        for s in range(GL):
            for c in idx_copies(s, s):
                c.start()
        for s in range(GL):
            for c in idx_copies(s, s):
                c.wait()
            gather(s, s % _NR).start()
        for s in range(GL, IL):
            for c in idx_copies(s, s):
                c.start()

        @pl.loop(0, cpw // _NI)
        def _edges(go):
            for u in range(_NI):
                g = _NI * go + u
                sr = u % _NR

                gather(u, sr).wait()
                scatter(u, sr).start(add=True)

                @pl.when(g >= 1)
                def _():
                    scatter((u - 1) % _NI, (u - 1) % _NR).wait()

                @pl.when(g + GL < cpw)
                def _():
                    for c in idx_copies(g + GL, (u + GL) % _NI):
                        c.wait()
                    gather((u + GL) % _NI, (u + GL) % _NR).start()

                @pl.when(g + IL < cpw)
                def _():
                    for c in idx_copies(g + IL, (u + IL) % _NI):
                        c.start()

        scatter((cpw - 1) % _NI, (cpw - 1) % _NR).wait()

        @pl.when(wid < extra)
        def _tail():
            off = (_NW * cpw + wid) * _CH
            pltpu.sync_copy(src_hbm.at[pl.ds(off, _CH)], idxr.at[0])
            pltpu.sync_copy(dst_hbm.at[pl.ds(off, _CH)], idxr.at[_NI])
            pltpu.async_copy(x_hbm.at[idxr.at[0]], rows[0], gsem[0]).wait()
            pltpu.sync_copy(rows[0], acc_sh.at[idxr.at[_NI]], add=True)

        plsc.subcore_barrier()

        for t in range(passes):
            ch = t * _NS + sid
            @pl.when(ch < n_row_chunks)
            def _():
                pltpu.sync_copy(acc_sh.at[pl.ds(ch * CR, CR)],
                                out_hbm.at[cid, pl.ds(ch * CR, CR)])

    return agg


def _mlp_body(x_ref, p0_ref, p1_ref, w1_ref, b1_ref, w2_ref, b2_ref,
              h_ref, sum_ref, sq_ref):
    i = pl.program_id(0)
    s = x_ref[...] + p0_ref[...] + p1_ref[...]
    h = jnp.maximum(
        jnp.dot(s, w1_ref[...], preferred_element_type=jnp.float32) + b1_ref[...], 0.0)
    h = jnp.maximum(
        jnp.dot(h, w2_ref[...], preferred_element_type=jnp.float32) + b2_ref[...], 0.0)
    h_ref[...] = h

    @pl.when(i == 0)
    def _():
        sum_ref[...] = jnp.zeros_like(sum_ref)
        sq_ref[...] = jnp.zeros_like(sq_ref)

    sum_ref[0:1, :] = sum_ref[0:1, :] + jnp.sum(h, axis=0, keepdims=True)
    sq_ref[0:1, :] = sq_ref[0:1, :] + jnp.sum(h * h, axis=0, keepdims=True)


def _bn_body(n_rows, h_ref, sum_ref, sq_ref, g_ref, b_ref, o_ref):
    mean = sum_ref[0:1, :] / n_rows
    var = sq_ref[0:1, :] / n_rows - mean * mean
    rstd = lax.rsqrt(var + 1e-5)
    o_ref[...] = (h_ref[...] - mean) * rstd * g_ref[...] + b_ref[...]


def kernel(x, edge_index, gamma, beta,
           W1_0, b1_0, W2_0, b2_0,
           W1_1, b1_1, W2_1, b2_1,
           W1_2, b1_2, W2_2, b2_2):
    N, D = x.shape
    E = edge_index.shape[1]
    H = W1_0.shape[1]
    n_chunks = E // _CH
    cpw = n_chunks // _NW
    n_main = _NW * cpw

    src_1d = edge_index[0]
    dst_1d = edge_index[1]
    zeros_h = jnp.zeros((_CR, D), jnp.float32)

    agg = _make_agg(N, E, D)

    NB = 10
    B = N // NB
    row_spec = pl.BlockSpec((B, D), lambda i: (i, 0))
    full = lambda shape: pl.BlockSpec(shape, lambda i: (0, 0))

    mlp_call = pl.pallas_call(
        _mlp_body,
        grid=(NB,),
        in_specs=[row_spec, row_spec, row_spec,
                  full((D, H)), full((1, H)), full((H, H)), full((1, H))],
        out_specs=[pl.BlockSpec((B, H), lambda i: (i, 0)),
                   full((8, H)), full((8, H))],
        out_shape=[jax.ShapeDtypeStruct((N, H), jnp.float32),
                   jax.ShapeDtypeStruct((8, H), jnp.float32),
                   jax.ShapeDtypeStruct((8, H), jnp.float32)],
    )
    bn_call = pl.pallas_call(
        functools.partial(_bn_body, float(N)),
        grid=(NB,),
        in_specs=[pl.BlockSpec((B, H), lambda i: (i, 0)),
                  full((8, H)), full((8, H)), full((1, H)), full((1, H))],
        out_specs=pl.BlockSpec((B, H), lambda i: (i, 0)),
        out_shape=jax.ShapeDtypeStruct((N, H), jnp.float32),
    )

    g_r = gamma.reshape(1, H)
    be_r = beta.reshape(1, H)
    params = [(W1_0, b1_0, W2_0, b2_0),
              (W1_1, b1_1, W2_1, b2_1),
              (W1_2, b1_2, W2_2, b2_2)]

    cur = x
    for (W1, b1, W2, b2) in params:
        parts = agg(cur, src_1d, dst_1d, zeros_h)
        h, ssum, ssq = mlp_call(cur, parts[0], parts[1],
                                W1, b1.reshape(1, H), W2, b2.reshape(1, H))
        cur = bn_call(h, ssum, ssq, g_r, be_r)
    return cur


# speedup vs baseline: 1.2495x; 1.0200x over previous
import functools

import jax
import jax.numpy as jnp
from jax import lax
from jax.experimental import pallas as pl
from jax.experimental.pallas import tpu as pltpu
from jax.experimental.pallas import tpu_sc as plsc

_NC = 2
_NS = 16
_NW = _NC * _NS
_CH = 32
_CR = 400
_NR = 6
_NI = 12


def _make_agg(N, E, D):
    n_chunks = E // _CH
    cpw = n_chunks // _NW
    extra = n_chunks - cpw * _NW
    CR = _CR
    n_row_chunks = N // CR
    passes = (n_row_chunks + _NS - 1) // _NS
    mesh = plsc.VectorSubcoreMesh(core_axis_name="c", subcore_axis_name="s")

    @functools.partial(
        pl.kernel,
        out_type=jax.ShapeDtypeStruct((_NC, N, D), jnp.float32),
        mesh=mesh,
        scratch_types=[
            pltpu.VMEM((2 * _NI, _CH), jnp.int32),
            [pltpu.VMEM((_CH, D), jnp.float32)] * _NR,
            pltpu.VMEM_SHARED((N, D), jnp.float32),
            [pltpu.SemaphoreType.DMA] * _NR,
            [pltpu.SemaphoreType.DMA] * _NR,
            [pltpu.SemaphoreType.DMA] * _NI,
        ],
    )
    def agg(x_hbm, src_hbm, dst_hbm, zeros_hbm, out_hbm,
            idxr, rows, acc_sh, gsem, ssem, isem):
        cid = lax.axis_index("c")
        sid = lax.axis_index("s")
        wid = sid * _NC + cid

        for t in range(passes):
            ch = t * _NS + sid
            @pl.when(ch < n_row_chunks)
            def _():
                pltpu.sync_copy(zeros_hbm, acc_sh.at[pl.ds(ch * CR, CR)])

        plsc.subcore_barrier()

        e0 = wid * cpw
        GL = _NR - 1
        IL = _NI - 1

        def idx_copies(chunk, si):
            off = (e0 + chunk) * _CH
            return (pltpu.make_async_copy(src_hbm.at[pl.ds(off, _CH)],
                                          idxr.at[si], isem[si]),
                    pltpu.make_async_copy(dst_hbm.at[pl.ds(off, _CH)],
                                          idxr.at[_NI + si], isem[si]))

        def gather(si, sr):
            return pltpu.make_async_copy(x_hbm.at[idxr.at[si]],
                                         rows[sr], gsem[sr])

        def scatter(si, sr):
            return pltpu.make_async_copy(rows[sr],
                                         acc_sh.at[idxr.at[_NI + si]], ssem[sr])

        for s in range(GL):
            for c in idx_copies(s, s):
                c.start()
        for s in range(GL):
            for c in idx_copies(s, s):
                c.wait()
            gather(s, s % _NR).start()
        for s in range(GL, IL):
            for c in idx_copies(s, s):
                c.start()

        @pl.loop(0, cpw // _NI)
        def _edges(go):
            for u in range(_NI):
                g = _NI * go + u
                sr = u % _NR

                gather(u, sr).wait()
                scatter(u, sr).start(add=True)

                @pl.when(g >= 1)
                def _():
                    scatter((u - 1) % _NI, (u - 1) % _NR).wait()

                @pl.when(g + GL < cpw)
                def _():
                    for c in idx_copies(g + GL, (u + GL) % _NI):
                        c.wait()
                    gather((u + GL) % _NI, (u + GL) % _NR).start()

                @pl.when(g + IL < cpw)
                def _():
                    for c in idx_copies(g + IL, (u + IL) % _NI):
                        c.start()

        scatter((cpw - 1) % _NI, (cpw - 1) % _NR).wait()

        @pl.when(wid < extra)
        def _tail():
            off = (_NW * cpw + wid) * _CH
            pltpu.sync_copy(src_hbm.at[pl.ds(off, _CH)], idxr.at[0])
            pltpu.sync_copy(dst_hbm.at[pl.ds(off, _CH)], idxr.at[_NI])
            pltpu.async_copy(x_hbm.at[idxr.at[0]], rows[0], gsem[0]).wait()
            pltpu.sync_copy(rows[0], acc_sh.at[idxr.at[_NI]], add=True)

        plsc.subcore_barrier()

        for t in range(passes):
            ch = t * _NS + sid
            @pl.when(ch < n_row_chunks)
            def _():
                pltpu.sync_copy(acc_sh.at[pl.ds(ch * CR, CR)],
                                out_hbm.at[cid, pl.ds(ch * CR, CR)])

    return agg


def _layer_body(nb, n_rows, blk,
                x_ref, p0_ref, p1_ref, w1_ref, b1_ref, w2_ref, b2_ref,
                g_ref, be_ref, o_ref, hbuf, sum_ref, sq_ref):
    i = pl.program_id(0)

    @pl.when(i == 0)
    def _():
        sum_ref[...] = jnp.zeros_like(sum_ref)
        sq_ref[...] = jnp.zeros_like(sq_ref)

    @pl.when(i < nb)
    def _():
        s = x_ref[...] + p0_ref[...] + p1_ref[...]
        h = jnp.maximum(
            jnp.dot(s, w1_ref[...], preferred_element_type=jnp.float32)
            + b1_ref[...], 0.0)
        h = jnp.maximum(
            jnp.dot(h, w2_ref[...], preferred_element_type=jnp.float32)
            + b2_ref[...], 0.0)
        hbuf[pl.ds(i * blk, blk), :] = h
        sum_ref[0:1, :] = sum_ref[0:1, :] + jnp.sum(h, axis=0, keepdims=True)
        sq_ref[0:1, :] = sq_ref[0:1, :] + jnp.sum(h * h, axis=0, keepdims=True)

    @pl.when(i >= nb)
    def _():
        j = i - nb
        mean = sum_ref[0:1, :] / n_rows
        var = sq_ref[0:1, :] / n_rows - mean * mean
        rstd = lax.rsqrt(var + 1e-5)
        h = hbuf[pl.ds(j * blk, blk), :]
        o_ref[...] = (h - mean) * rstd * g_ref[...] + be_ref[...]


def kernel(x, edge_index, gamma, beta,
           W1_0, b1_0, W2_0, b2_0,
           W1_1, b1_1, W2_1, b2_1,
           W1_2, b1_2, W2_2, b2_2):
    N, D = x.shape
    E = edge_index.shape[1]
    H = W1_0.shape[1]
    n_chunks = E // _CH
    cpw = n_chunks // _NW
    n_main = _NW * cpw

    src_1d = edge_index[0]
    dst_1d = edge_index[1]
    zeros_h = jnp.zeros((_CR, D), jnp.float32)

    agg = _make_agg(N, E, D)

    NB = 10
    B = N // NB
    in_blk = pl.BlockSpec((B, D), lambda i: (jnp.where(i < NB, i, 0), 0))
    full = lambda shape: pl.BlockSpec(shape, lambda i: (0, 0))

    layer_call = pl.pallas_call(
        functools.partial(_layer_body, NB, float(N), B),
        grid=(2 * NB,),
        in_specs=[in_blk, in_blk, in_blk,
                  full((D, H)), full((1, H)), full((H, H)), full((1, H)),
                  full((1, H)), full((1, H))],
        out_specs=pl.BlockSpec((B, H),
                               lambda i: (jnp.where(i < NB, NB, i - NB), 0)),
        out_shape=jax.ShapeDtypeStruct((N + B, H), jnp.float32),
        scratch_shapes=[pltpu.VMEM((N, H), jnp.float32),
                        pltpu.VMEM((8, H), jnp.float32),
                        pltpu.VMEM((8, H), jnp.float32)],
    )

    g_r = gamma.reshape(1, H)
    be_r = beta.reshape(1, H)
    params = [(W1_0, b1_0, W2_0, b2_0),
              (W1_1, b1_1, W2_1, b2_1),
              (W1_2, b1_2, W2_2, b2_2)]

    cur = x
    for (W1, b1, W2, b2) in params:
        parts = agg(cur, src_1d, dst_1d, zeros_h)
        cur = layer_call(cur, parts[0], parts[1],
                         W1, b1.reshape(1, H), W2, b2.reshape(1, H),
                         g_r, be_r)[:N]
    return cur


# speedup vs baseline: 1.2534x; 1.0031x over previous
import functools

import jax
import jax.numpy as jnp
from jax import lax
from jax.experimental import pallas as pl
from jax.experimental.pallas import tpu as pltpu
from jax.experimental.pallas import tpu_sc as plsc

_NC = 2
_NS = 16
_NW = _NC * _NS
_CH = 32
_CR = 400
_NR = 6
_NI = 12


def _make_agg(N, E, D):
    n_chunks = E // _CH
    cpw = n_chunks // _NW
    extra = n_chunks - cpw * _NW
    CR = _CR
    n_row_chunks = N // CR
    passes = (n_row_chunks + _NS - 1) // _NS
    mesh = plsc.VectorSubcoreMesh(core_axis_name="c", subcore_axis_name="s")

    @functools.partial(
        pl.kernel,
        out_type=jax.ShapeDtypeStruct((_NC, N, D), jnp.float32),
        mesh=mesh,
        scratch_types=[
            pltpu.VMEM((2 * _NI, _CH), jnp.int32),
            [pltpu.VMEM((_CH, D), jnp.float32)] * _NR,
            pltpu.VMEM_SHARED((N, D), jnp.float32),
            [pltpu.SemaphoreType.DMA] * _NR,
            [pltpu.SemaphoreType.DMA] * _NR,
            [pltpu.SemaphoreType.DMA] * _NI,
        ],
    )
    def agg(x_hbm, src_hbm, dst_hbm, zeros_hbm, out_hbm,
            idxr, rows, acc_sh, gsem, ssem, isem):
        cid = lax.axis_index("c")
        sid = lax.axis_index("s")
        wid = sid * _NC + cid

        e0 = wid * cpw
        GL = _NR - 1
        IL = _NI - 1

        def idx_copies(chunk, si):
            off = (e0 + chunk) * _CH
            return (pltpu.make_async_copy(src_hbm.at[pl.ds(off, _CH)],
                                          idxr.at[si], isem[si]),
                    pltpu.make_async_copy(dst_hbm.at[pl.ds(off, _CH)],
                                          idxr.at[_NI + si], isem[si]))

        def gather(si, sr):
            return pltpu.make_async_copy(x_hbm.at[idxr.at[si]],
                                         rows[sr], gsem[sr])

        def scatter(si, sr):
            return pltpu.make_async_copy(rows[sr],
                                         acc_sh.at[idxr.at[_NI + si]], ssem[sr])

        for s in range(GL):
            for c in idx_copies(s, s):
                c.start()
        for s in range(GL):
            for c in idx_copies(s, s):
                c.wait()
            gather(s, s % _NR).start()
        for s in range(GL, IL):
            for c in idx_copies(s, s):
                c.start()

        for t in range(passes):
            ch = t * _NS + sid
            @pl.when(ch < n_row_chunks)
            def _():
                pltpu.sync_copy(zeros_hbm, acc_sh.at[pl.ds(ch * CR, CR)])

        plsc.subcore_barrier()

        @pl.loop(0, cpw // _NI)
        def _edges(go):
            for u in range(_NI):
                g = _NI * go + u
                sr = u % _NR

                gather(u, sr).wait()
                scatter(u, sr).start(add=True)

                @pl.when(g >= 1)
                def _():
                    scatter((u - 1) % _NI, (u - 1) % _NR).wait()

                @pl.when(g + GL < cpw)
                def _():
                    for c in idx_copies(g + GL, (u + GL) % _NI):
                        c.wait()
                    gather((u + GL) % _NI, (u + GL) % _NR).start()

                @pl.when(g + IL < cpw)
                def _():
                    for c in idx_copies(g + IL, (u + IL) % _NI):
                        c.start()

        scatter((cpw - 1) % _NI, (cpw - 1) % _NR).wait()

        @pl.when(wid < extra)
        def _tail():
            off = (_NW * cpw + wid) * _CH
            pltpu.sync_copy(src_hbm.at[pl.ds(off, _CH)], idxr.at[0])
            pltpu.sync_copy(dst_hbm.at[pl.ds(off, _CH)], idxr.at[_NI])
            pltpu.async_copy(x_hbm.at[idxr.at[0]], rows[0], gsem[0]).wait()
            pltpu.sync_copy(rows[0], acc_sh.at[idxr.at[_NI]], add=True)

        plsc.subcore_barrier()

        for t in range(passes):
            ch = t * _NS + sid
            @pl.when(ch < n_row_chunks)
            def _():
                pltpu.sync_copy(acc_sh.at[pl.ds(ch * CR, CR)],
                                out_hbm.at[cid, pl.ds(ch * CR, CR)])

    return agg


def _layer_body(nb, n_rows, blk,
                x_ref, p0_ref, p1_ref, w1_ref, b1_ref, w2_ref, b2_ref,
                g_ref, be_ref, o_ref, hbuf, sum_ref, sq_ref):
    i = pl.program_id(0)

    @pl.when(i == 0)
    def _():
        sum_ref[...] = jnp.zeros_like(sum_ref)
        sq_ref[...] = jnp.zeros_like(sq_ref)

    @pl.when(i < nb)
    def _():
        s = x_ref[...] + p0_ref[...] + p1_ref[...]
        h = jnp.maximum(
            jnp.dot(s, w1_ref[...], preferred_element_type=jnp.float32)
            + b1_ref[...], 0.0)
        h = jnp.maximum(
            jnp.dot(h, w2_ref[...], preferred_element_type=jnp.float32)
            + b2_ref[...], 0.0)
        hbuf[pl.ds(i * blk, blk), :] = h
        sum_ref[0:1, :] = sum_ref[0:1, :] + jnp.sum(h, axis=0, keepdims=True)
        sq_ref[0:1, :] = sq_ref[0:1, :] + jnp.sum(h * h, axis=0, keepdims=True)

    @pl.when(i >= nb)
    def _():
        j = i - nb
        mean = sum_ref[0:1, :] / n_rows
        var = sq_ref[0:1, :] / n_rows - mean * mean
        rstd = lax.rsqrt(var + 1e-5)
        h = hbuf[pl.ds(j * blk, blk), :]
        o_ref[...] = (h - mean) * rstd * g_ref[...] + be_ref[...]


def kernel(x, edge_index, gamma, beta,
           W1_0, b1_0, W2_0, b2_0,
           W1_1, b1_1, W2_1, b2_1,
           W1_2, b1_2, W2_2, b2_2):
    N, D = x.shape
    E = edge_index.shape[1]
    H = W1_0.shape[1]
    n_chunks = E // _CH
    cpw = n_chunks // _NW
    n_main = _NW * cpw

    src_1d = edge_index[0]
    dst_1d = edge_index[1]
    zeros_h = jnp.zeros((_CR, D), jnp.float32)

    agg = _make_agg(N, E, D)

    NB = 10
    B = N // NB
    in_blk = pl.BlockSpec((B, D), lambda i: (jnp.where(i < NB, i, 0), 0))
    full = lambda shape: pl.BlockSpec(shape, lambda i: (0, 0))

    layer_call = pl.pallas_call(
        functools.partial(_layer_body, NB, float(N), B),
        grid=(2 * NB,),
        in_specs=[in_blk, in_blk, in_blk,
                  full((D, H)), full((1, H)), full((H, H)), full((1, H)),
                  full((1, H)), full((1, H))],
        out_specs=pl.BlockSpec((B, H),
                               lambda i: (jnp.where(i < NB, NB, i - NB), 0)),
        out_shape=jax.ShapeDtypeStruct((N + B, H), jnp.float32),
        scratch_shapes=[pltpu.VMEM((N, H), jnp.float32),
                        pltpu.VMEM((8, H), jnp.float32),
                        pltpu.VMEM((8, H), jnp.float32)],
    )

    g_r = gamma.reshape(1, H)
    be_r = beta.reshape(1, H)
    params = [(W1_0, b1_0, W2_0, b2_0),
              (W1_1, b1_1, W2_1, b2_1),
              (W1_2, b1_2, W2_2, b2_2)]

    cur = x
    for (W1, b1, W2, b2) in params:
        parts = agg(cur, src_1d, dst_1d, zeros_h)
        cur = layer_call(cur, parts[0], parts[1],
                         W1, b1.reshape(1, H), W2, b2.reshape(1, H),
                         g_r, be_r)[:N]
    return cur
